# baseline (speedup 1.0000x reference)
import functools

import numpy as np
import jax
import jax.numpy as jnp
from jax import lax
from jax.experimental import pallas as pl
from jax.experimental.pallas import tpu as pltpu
from jax.experimental.pallas import tpu_sc as plsc

N = 10000
E = 320000
D = 128
DP = D // 2

NC = 2
NS = 16
NW = NC * NS
EDGES_PER_TILE = E // NW
CHUNK = 80
N_CHUNKS = EDGES_PER_TILE // CHUNK
NB = 4
NSB = 2
NI = 8
STRIPE = 624

_PERM = np.arange(D).reshape(D // 32, 16, 2).transpose(0, 2, 1).reshape(D)


def _sc_aggregate(xp, src, dst, edge_weight):
    mesh = plsc.VectorSubcoreMesh(core_axis_name="c", subcore_axis_name="s")

    @functools.partial(
        pl.kernel,
        out_type=jax.ShapeDtypeStruct((NC, N, D), jnp.float32),
        mesh=mesh,
        compiler_params=pltpu.CompilerParams(use_tc_tiling_on_sc=False),
        scratch_types=[
            pltpu.VMEM((NI, CHUNK), jnp.int32),
            pltpu.VMEM((NI, CHUNK), jnp.int32),
            pltpu.VMEM((NI, CHUNK), jnp.float32),
            [pltpu.VMEM((CHUNK, DP), jnp.int32)] * NB,
            [pltpu.VMEM((CHUNK, D), jnp.float32)] * NSB,
            pltpu.VMEM_SHARED((N, D), jnp.float32),
            [pltpu.SemaphoreType.DMA] * NI,
            [pltpu.SemaphoreType.DMA] * NB,
            [pltpu.SemaphoreType.DMA] * NSB,
        ],
    )
    def agg(x_hbm, src_hbm, dst_hbm, ew_hbm, out_hbm,
            src_v, dst_v, w_v, gbufs, sbufs, acc, isems, gsems, ssems):
        c = lax.axis_index("c")
        s = lax.axis_index("s")
        wid = c * NS + s
        base = wid * EDGES_PER_TILE

        def stage_idx(g, slot):
            off = base + g * CHUNK
            pltpu.async_copy(src_hbm.at[pl.ds(off, CHUNK)],
                             src_v.at[slot], isems[slot])
            pltpu.async_copy(dst_hbm.at[pl.ds(off, CHUNK)],
                             dst_v.at[slot], isems[slot])
            pltpu.async_copy(ew_hbm.at[pl.ds(off, CHUNK)],
                             w_v.at[slot], isems[slot])

        def wait_idx(g, slot):
            off = base + g * CHUNK
            pltpu.make_async_copy(src_hbm.at[pl.ds(off, CHUNK)],
                                  src_v.at[slot], isems[slot]).wait()
            pltpu.make_async_copy(dst_hbm.at[pl.ds(off, CHUNK)],
                                  dst_v.at[slot], isems[slot]).wait()
            pltpu.make_async_copy(ew_hbm.at[pl.ds(off, CHUNK)],
                                  w_v.at[slot], isems[slot]).wait()

        def issue_gather(slot, b):
            pltpu.async_copy(x_hbm.at[src_v.at[slot]], gbufs[b], gsems[b])

        def wait_gather(slot, b):
            pltpu.make_async_copy(
                x_hbm.at[src_v.at[slot]], gbufs[b], gsems[b]).wait()

        def issue_scatter(slot, o):
            pltpu.async_copy(sbufs[o], acc.at[dst_v.at[slot]], ssems[o],
                             add=True)

        def wait_scatter(slot, o):
            pltpu.make_async_copy(
                sbufs[o], acc.at[dst_v.at[slot]], ssems[o]).wait()

        for g in range(NI - 1):
            stage_idx(g, g)

        zvec = jnp.zeros((16,), jnp.float32)

        def zero_row(r, _):
            for j in range(D // 16):
                sbufs[0][r, pl.ds(j * 16, 16)] = zvec
            return 0

        lax.fori_loop(0, CHUNK, zero_row, 0)
        row0 = s * STRIPE
        for k in range(STRIPE // CHUNK):
            pltpu.sync_copy(sbufs[0], acc.at[pl.ds(row0 + k * CHUNK, CHUNK)])
        nfull = (STRIPE // CHUNK) * CHUNK
        pltpu.sync_copy(sbufs[0].at[pl.ds(0, STRIPE - nfull)],
                        acc.at[pl.ds(row0 + nfull, STRIPE - nfull)])

        @pl.when(s == NS - 1)
        def _():
            pltpu.sync_copy(sbufs[0].at[pl.ds(0, N - NS * STRIPE)],
                            acc.at[pl.ds(NS * STRIPE, N - NS * STRIPE)])

        himask = jnp.full((16,), -65536, jnp.int32)

        def scale(b, o, slot):
            def grp(k16, _):
                wv = w_v[slot, pl.ds(k16 * 16, 16)]
                for l in range(16):
                    wbc = jnp.full((16,), wv[l], jnp.float32)
                    e = k16 * 16 + l
                    for j in range(D // 32):
                        iv = gbufs[b][e, pl.ds(j * 16, 16)]
                        lo = lax.bitcast_convert_type(iv << 16, jnp.float32)
                        hi = lax.bitcast_convert_type(iv & himask,
                                                      jnp.float32)
                        sbufs[o][e, pl.ds(j * 32, 16)] = lo * wbc
                        sbufs[o][e, pl.ds(j * 32 + 16, 16)] = hi * wbc
                return 0

            lax.fori_loop(0, CHUNK // 16, grp, 0)

        for g in range(NB - 1):
            wait_idx(g, g)
            issue_gather(g, g)

        plsc.subcore_barrier()

        def when(cond, fn):
            if isinstance(cond, bool):
                if cond:
                    fn()
            else:
                pl.when(cond)(fn)

        def process(g, bi, si, oi):
            wait_gather(si, bi)
            scale(bi, oi, si)

            when(g >= 1,
                 lambda: wait_scatter((si - 1) % NI, (oi - 1) % NSB))
            when(g + NI - 1 < N_CHUNKS,
                 lambda: stage_idx(g + NI - 1, (si - 1) % NI))

            def prefetch():
                wait_idx(g + NB - 1, (si + NB - 1) % NI)
                issue_gather((si + NB - 1) % NI, (bi - 1) % NB)

            when(g + NB - 1 < N_CHUNKS, prefetch)
            issue_scatter(si, oi)

        def octet(k, _):
            for b in range(NI):
                g = NI * k + b
                process(g, b % NB, b, b % NSB)
            return 0

        lax.fori_loop(0, N_CHUNKS // NI, octet, 0)
        for b in range(N_CHUNKS % NI):
            g = (N_CHUNKS // NI) * NI + b
            process(g, g % NB, g % NI, g % NSB)

        wait_scatter((N_CHUNKS - 1) % NI, (N_CHUNKS - 1) % NSB)

        plsc.subcore_barrier()

        pltpu.sync_copy(acc.at[pl.ds(row0, STRIPE)],
                        out_hbm.at[c, pl.ds(row0, STRIPE)])

        @pl.when(s == NS - 1)
        def _():
            pltpu.sync_copy(acc.at[pl.ds(NS * STRIPE, N - NS * STRIPE)],
                            out_hbm.at[c, pl.ds(NS * STRIPE,
                                                N - NS * STRIPE)])

    return agg(xp, src, dst, edge_weight)


def _mm_body(p_ref, w_ref, o_ref):
    s = p_ref[0] + p_ref[1]
    o_ref[...] = jnp.dot(s, w_ref[...], preferred_element_type=jnp.float32)


BM = 400


def _tc_matmul(partials, W):
    return pl.pallas_call(
        _mm_body,
        grid=(N // BM,),
        in_specs=[
            pl.BlockSpec((NC, BM, D), lambda i: (0, i, 0)),
            pl.BlockSpec((D, D), lambda i: (0, 0)),
        ],
        out_specs=pl.BlockSpec((BM, D), lambda i: (i, 0)),
        out_shape=jax.ShapeDtypeStruct((N, D), jnp.float32),
    )(partials, W)


@jax.jit
def kernel(x, edge_index, edge_weight, W):
    xp = lax.bitcast_convert_type(
        x.astype(jnp.bfloat16).reshape(N, DP, 2), jnp.int32)
    partials = _sc_aggregate(xp, edge_index[0], edge_index[1], edge_weight)
    return _tc_matmul(partials, W[_PERM, :])


# speedup vs baseline: 2.0755x; 2.0755x over previous
import functools

import jax
import jax.numpy as jnp
from jax import lax
from jax.experimental import pallas as pl
from jax.experimental.pallas import tpu as pltpu
from jax.experimental.pallas import tpu_sc as plsc

N = 10000
E = 320000
D = 128

NC = 2
NS = 16
NW = NC * NS
EDGES_PER_TILE = E // NW
CHUNK = 80
N_CHUNKS = EDGES_PER_TILE // CHUNK
NB = 4
NI = 8
STRIPE = 624


def _sc_aggregate(x, src, dst, edge_weight):
    mesh = plsc.VectorSubcoreMesh(core_axis_name="c", subcore_axis_name="s")

    @functools.partial(
        pl.kernel,
        out_type=jax.ShapeDtypeStruct((NC, N, D), jnp.float32),
        mesh=mesh,
        scratch_types=[
            pltpu.VMEM((NI, CHUNK), jnp.int32),
            pltpu.VMEM((NI, CHUNK), jnp.int32),
            pltpu.VMEM((NI, CHUNK), jnp.float32),
            [pltpu.VMEM((CHUNK, D), jnp.float32)] * NB,
            pltpu.VMEM_SHARED((N, D), jnp.float32),
            [pltpu.SemaphoreType.DMA] * NI,
            [pltpu.SemaphoreType.DMA] * NB,
            [pltpu.SemaphoreType.DMA] * NB,
        ],
    )
    def agg(x_hbm, src_hbm, dst_hbm, ew_hbm, out_hbm,
            src_v, dst_v, w_v, bufs, acc, isems, gsems, ssems):
        c = lax.axis_index("c")
        s = lax.axis_index("s")
        wid = c * NS + s
        base = wid * EDGES_PER_TILE

        def stage_idx(g, slot):
            off = base + g * CHUNK
            pltpu.async_copy(src_hbm.at[pl.ds(off, CHUNK)],
                             src_v.at[slot], isems[slot])
            pltpu.async_copy(dst_hbm.at[pl.ds(off, CHUNK)],
                             dst_v.at[slot], isems[slot])
            pltpu.async_copy(ew_hbm.at[pl.ds(off, CHUNK)],
                             w_v.at[slot], isems[slot])

        def wait_idx(g, slot):
            off = base + g * CHUNK
            pltpu.make_async_copy(src_hbm.at[pl.ds(off, CHUNK)],
                                  src_v.at[slot], isems[slot]).wait()
            pltpu.make_async_copy(dst_hbm.at[pl.ds(off, CHUNK)],
                                  dst_v.at[slot], isems[slot]).wait()
            pltpu.make_async_copy(ew_hbm.at[pl.ds(off, CHUNK)],
                                  w_v.at[slot], isems[slot]).wait()

        def issue_gather(slot, b):
            pltpu.async_copy(x_hbm.at[src_v.at[slot]], bufs[b], gsems[b])

        def wait_gather(slot, b):
            pltpu.make_async_copy(
                x_hbm.at[src_v.at[slot]], bufs[b], gsems[b]).wait()

        def issue_scatter(slot, b):
            pltpu.async_copy(bufs[b], acc.at[dst_v.at[slot]], ssems[b],
                             add=True)

        def wait_scatter(slot, b):
            pltpu.make_async_copy(
                bufs[b], acc.at[dst_v.at[slot]], ssems[b]).wait()

        for g in range(NI - 1):
            stage_idx(g, g)

        zvec = jnp.zeros((16,), jnp.float32)

        def zero_row(r, _):
            for j in range(D // 16):
                bufs[0][r, pl.ds(j * 16, 16)] = zvec
            return 0

        lax.fori_loop(0, CHUNK, zero_row, 0)
        row0 = s * STRIPE
        for k in range(STRIPE // CHUNK):
            pltpu.sync_copy(bufs[0], acc.at[pl.ds(row0 + k * CHUNK, CHUNK)])
        nfull = (STRIPE // CHUNK) * CHUNK
        pltpu.sync_copy(bufs[0].at[pl.ds(0, STRIPE - nfull)],
                        acc.at[pl.ds(row0 + nfull, STRIPE - nfull)])

        @pl.when(s == NS - 1)
        def _():
            pltpu.sync_copy(bufs[0].at[pl.ds(0, N - NS * STRIPE)],
                            acc.at[pl.ds(NS * STRIPE, N - NS * STRIPE)])

        def scale(b, slot):
            def grp(k16, _):
                wv = w_v[slot, pl.ds(k16 * 16, 16)]
                for l in range(16):
                    wbc = jnp.full((16,), wv[l], jnp.float32)
                    e = k16 * 16 + l
                    for j in range(D // 16):
                        seg = bufs[b][e, pl.ds(j * 16, 16)]
                        bufs[b][e, pl.ds(j * 16, 16)] = seg * wbc
                return 0

            lax.fori_loop(0, CHUNK // 16, grp, 0)

        for g in range(NB - 1):
            wait_idx(g, g)
            issue_gather(g, g)

        plsc.subcore_barrier()

        def when(cond, fn):
            if isinstance(cond, bool):
                if cond:
                    fn()
            else:
                pl.when(cond)(fn)

        def process(g, bi, si):
            wait_gather(si, bi)
            scale(bi, si)

            when(g >= 1,
                 lambda: wait_scatter((si - 1) % NI, (bi - 1) % NB))
            when(g + NI - 1 < N_CHUNKS,
                 lambda: stage_idx(g + NI - 1, (si - 1) % NI))

            def prefetch():
                wait_idx(g + NB - 1, (si + NB - 1) % NI)
                issue_gather((si + NB - 1) % NI, (bi - 1) % NB)

            when(g + NB - 1 < N_CHUNKS, prefetch)
            issue_scatter(si, bi)

        def octet(k, _):
            for b in range(NI):
                g = NI * k + b
                process(g, b % NB, b)
            return 0

        lax.fori_loop(0, N_CHUNKS // NI, octet, 0)
        for b in range(N_CHUNKS % NI):
            g = (N_CHUNKS // NI) * NI + b
            process(g, g % NB, g % NI)

        wait_scatter((N_CHUNKS - 1) % NI, (N_CHUNKS - 1) % NB)

        plsc.subcore_barrier()

        pltpu.sync_copy(acc.at[pl.ds(row0, STRIPE)],
                        out_hbm.at[c, pl.ds(row0, STRIPE)])

        @pl.when(s == NS - 1)
        def _():
            pltpu.sync_copy(acc.at[pl.ds(NS * STRIPE, N - NS * STRIPE)],
                            out_hbm.at[c, pl.ds(NS * STRIPE,
                                                N - NS * STRIPE)])

    return agg(x, src, dst, edge_weight)


def _mm_body(p_ref, w_ref, o_ref):
    s = p_ref[0] + p_ref[1]
    o_ref[...] = jnp.dot(s, w_ref[...], preferred_element_type=jnp.float32)


BM = 400


def _tc_matmul(partials, W):
    return pl.pallas_call(
        _mm_body,
        grid=(N // BM,),
        in_specs=[
            pl.BlockSpec((NC, BM, D), lambda i: (0, i, 0)),
            pl.BlockSpec((D, D), lambda i: (0, 0)),
        ],
        out_specs=pl.BlockSpec((BM, D), lambda i: (i, 0)),
        out_shape=jax.ShapeDtypeStruct((N, D), jnp.float32),
    )(partials, W)


@jax.jit
def kernel(x, edge_index, edge_weight, W):
    partials = _sc_aggregate(x, edge_index[0], edge_index[1], edge_weight)
    return _tc_matmul(partials, W)
